# baseline (speedup 1.0000x reference)
import jax
import jax.numpy as jnp
from jax.experimental import pallas as pl
from jax.experimental.pallas import tpu as pltpu

_N = 16384
_K = 1024
_DIM = 256
_BN = 4096
_BC = 1024
_AUG = _DIM + 2
_STEPS = _N // _BN


def _cdist_mean_kernel(x_ref, c_ref, out_ref, acc_ref, cb_ref):
    i = pl.program_id(0)

    @pl.when(i == 0)
    def _init():
        cf = c_ref[...]
        csq = (cf * cf).astype(jnp.bfloat16)
        c2c = jax.lax.dot_general(
            csq,
            jnp.ones((8, _DIM), jnp.bfloat16),
            dimension_numbers=(((1,), (1,)), ((), ())),
            preferred_element_type=jnp.float32,
        )[:, :1]
        cb_ref[...] = jnp.concatenate(
            [
                (cf * -2.0).astype(jnp.bfloat16),
                jnp.ones((_K, 1), jnp.bfloat16),
                c2c.astype(jnp.bfloat16),
            ],
            axis=1,
        )
        acc_ref[...] = jnp.zeros((8, _K), jnp.float32)

    cb = cb_ref[...]
    pacc = jnp.zeros((8, _K), jnp.float32)
    for c in range(_BN // _BC):
        xb = x_ref[pl.ds(c * _BC, _BC), :].astype(jnp.bfloat16)
        x2 = jax.lax.dot_general(
            xb * xb,
            jnp.ones((8, _DIM), jnp.bfloat16),
            dimension_numbers=(((1,), (1,)), ((), ())),
            preferred_element_type=jnp.float32,
        )[:, :1]
        xaug = jnp.concatenate(
            [xb, x2.astype(jnp.bfloat16), jnp.ones((_BC, 1), jnp.bfloat16)],
            axis=1,
        )
        d2 = jax.lax.dot_general(
            xaug,
            cb,
            dimension_numbers=(((1,), (1,)), ((), ())),
            preferred_element_type=jnp.float32,
        )
        dist = d2 * jax.lax.rsqrt(d2)
        pacc = pacc + jnp.sum(dist.reshape(_BC // 8, 8, _K), axis=0)
    acc_ref[...] += pacc

    @pl.when(i == _STEPS - 1)
    def _final():
        out_ref[...] = (jnp.sum(acc_ref[...]) * jnp.float32(1.0 / (_N * _K)))[
            None, None
        ]


@jax.jit
def kernel(x, codebook):
    out = pl.pallas_call(
        _cdist_mean_kernel,
        grid=(_STEPS,),
        in_specs=[
            pl.BlockSpec((_BN, _DIM), lambda i: (i, 0)),
            pl.BlockSpec((_K, _DIM), lambda i: (0, 0)),
        ],
        out_specs=pl.BlockSpec((1, 1), lambda i: (0, 0)),
        out_shape=jax.ShapeDtypeStruct((1, 1), jnp.float32),
        scratch_shapes=[
            pltpu.VMEM((8, _K), jnp.float32),
            pltpu.VMEM((_K, _AUG), jnp.bfloat16),
        ],
    )(x, codebook)
    return out[0, 0]


# speedup vs baseline: 1.4305x; 1.4305x over previous
import jax
import jax.numpy as jnp
from jax.experimental import pallas as pl
from jax.experimental.pallas import tpu as pltpu

_N = 16384
_K = 1024
_DIM = 256
_BN = 4096
_BC = 1024
_STEPS = _N // _BN


def _cdist_mean_kernel(x_ref, c_ref, out_ref, acc_ref, cb_ref, c2_ref):
    i = pl.program_id(0)

    @pl.when(i == 0)
    def _init():
        cf = c_ref[...]
        csq = (cf * cf).astype(jnp.bfloat16)
        c2r = jax.lax.dot_general(
            jnp.ones((8, _DIM), jnp.bfloat16),
            csq,
            dimension_numbers=(((1,), (1,)), ((), ())),
            preferred_element_type=jnp.float32,
        )
        cb_ref[...] = (cf * -2.0).astype(jnp.bfloat16)
        c2_ref[...] = c2r[:1, :]
        acc_ref[...] = jnp.zeros((8, _K), jnp.float32)

    c2 = c2_ref[...]
    cb = cb_ref[...]
    pacc = jnp.zeros((8, _K), jnp.float32)
    for c in range(_BN // _BC):
        xb = x_ref[pl.ds(c * _BC, _BC), :].astype(jnp.bfloat16)
        x2 = jax.lax.dot_general(
            xb * xb,
            jnp.ones((8, _DIM), jnp.bfloat16),
            dimension_numbers=(((1,), (1,)), ((), ())),
            preferred_element_type=jnp.float32,
        )[:, :1]
        dot = jax.lax.dot_general(
            xb,
            cb,
            dimension_numbers=(((1,), (1,)), ((), ())),
            preferred_element_type=jnp.float32,
        )
        d2 = dot + (x2 + c2)
        dist = d2 * jax.lax.rsqrt(d2)
        pacc = pacc + jnp.sum(dist.reshape(_BC // 8, 8, _K), axis=0)
    acc_ref[...] += pacc

    @pl.when(i == _STEPS - 1)
    def _final():
        out_ref[...] = (jnp.sum(acc_ref[...]) * jnp.float32(1.0 / (_N * _K)))[
            None, None
        ]


@jax.jit
def kernel(x, codebook):
    out = pl.pallas_call(
        _cdist_mean_kernel,
        grid=(_STEPS,),
        in_specs=[
            pl.BlockSpec((_BN, _DIM), lambda i: (i, 0)),
            pl.BlockSpec((_K, _DIM), lambda i: (0, 0)),
        ],
        out_specs=pl.BlockSpec((1, 1), lambda i: (0, 0)),
        out_shape=jax.ShapeDtypeStruct((1, 1), jnp.float32),
        scratch_shapes=[
            pltpu.VMEM((8, _K), jnp.float32),
            pltpu.VMEM((_K, _DIM), jnp.bfloat16),
            pltpu.VMEM((1, _K), jnp.float32),
        ],
    )(x, codebook)
    return out[0, 0]
